# baseline (speedup 1.0000x reference)
import functools
import math

import jax
import jax.numpy as jnp
from jax import lax
from jax.experimental import pallas as pl
from jax.experimental.pallas import tpu as pltpu
from jax.experimental.pallas import tpu_sc as plsc

_NUM_EXPERTS = 16
_LANES = 16


def _make_sc_fill(s: int, num_experts: int, capacity: int, k_inv: float):
    nc, ns = 2, 16
    nw = nc * ns
    total = num_experts * capacity
    per_w = total // nw
    assert per_w % _LANES == 0 and total == 2 * s
    assert capacity % per_w == 0

    mesh = plsc.VectorSubcoreMesh(
        core_axis_name="c", subcore_axis_name="s", num_cores=nc, num_subcores=ns
    )

    @functools.partial(
        pl.kernel,
        out_type=(
            jax.ShapeDtypeStruct((2 * s,), jnp.float32),
            jax.ShapeDtypeStruct((total,), jnp.int32),
        ),
        mesh=mesh,
        scratch_types=[
            pltpu.VMEM((per_w,), jnp.float32),
            pltpu.VMEM((per_w,), jnp.int32),
        ],
    )
    def fill(gates_hbm, mask_hbm, g_v, m_v):
        wid = lax.axis_index("s") * nc + lax.axis_index("c")
        base = wid * per_w
        e = base // capacity
        c0 = base - e * capacity
        lane = lax.iota(jnp.int32, 16)
        gfill = jnp.full((_LANES,), k_inv, dtype=jnp.float32)
        for j in range(per_w // _LANES):
            g_v[pl.ds(j * _LANES, _LANES)] = gfill
            m_v[pl.ds(j * _LANES, _LANES)] = (
                lane + (c0 + j * _LANES)
            ) * num_experts + e
        pltpu.sync_copy(g_v, gates_hbm.at[pl.ds(base, per_w)])
        pltpu.sync_copy(m_v, mask_hbm.at[pl.ds(base, per_w)])

    return fill


def kernel(input):
    s = int(input.shape[0])
    num_experts = _NUM_EXPERTS
    capacity_fp = 2 * s / num_experts
    capacity = int(math.ceil(capacity_fp))
    k = num_experts * capacity // s

    gates_flat, mask_flat = _make_sc_fill(s, num_experts, capacity, 1.0 / k)()
    gates = gates_flat.reshape(2, s)
    dispatch_mask = mask_flat.reshape(num_experts, capacity)
    return (gates, dispatch_mask, jnp.asarray(capacity_fp, dtype=jnp.float32))


# speedup vs baseline: 1.0117x; 1.0117x over previous
import functools
import math
import struct

import jax
import jax.numpy as jnp
from jax import lax
from jax.experimental import pallas as pl
from jax.experimental.pallas import tpu as pltpu
from jax.experimental.pallas import tpu_sc as plsc

_NUM_EXPERTS = 16
_LANES = 16


def _make_sc_fill(s: int, num_experts: int, capacity: int, k_inv: float):
    nc, ns = 2, 16
    nw = nc * ns
    total_mask = num_experts * capacity
    total = 2 * s + total_mask
    per_w = total // nw
    g_per_w = 2 * s // nw
    assert per_w % _LANES == 0 and g_per_w % _LANES == 0
    assert capacity % (per_w - g_per_w) == 0
    k_inv_bits = struct.unpack("<i", struct.pack("<f", k_inv))[0]

    mesh = plsc.VectorSubcoreMesh(
        core_axis_name="c", subcore_axis_name="s", num_cores=nc, num_subcores=ns
    )

    @functools.partial(
        pl.kernel,
        out_type=jax.ShapeDtypeStruct((total,), jnp.int32),
        mesh=mesh,
        scratch_types=[pltpu.VMEM((per_w,), jnp.int32)],
    )
    def fill(out_hbm, v):
        wid = lax.axis_index("s") * nc + lax.axis_index("c")
        lane = lax.iota(jnp.int32, 16)
        gsplat = jnp.full((_LANES,), k_inv_bits, dtype=jnp.int32)
        for j in range(g_per_w // _LANES):
            v[pl.ds(j * _LANES, _LANES)] = gsplat
        m_per_w = per_w - g_per_w
        mbase = wid * m_per_w
        e = mbase // capacity
        c0 = mbase - e * capacity
        for j in range(m_per_w // _LANES):
            v[pl.ds(g_per_w + j * _LANES, _LANES)] = (
                lane + (c0 + j * _LANES)
            ) * num_experts + e
        pltpu.sync_copy(
            v.at[pl.ds(0, g_per_w)], out_hbm.at[pl.ds(wid * g_per_w, g_per_w)]
        )
        pltpu.sync_copy(
            v.at[pl.ds(g_per_w, m_per_w)],
            out_hbm.at[pl.ds(2 * s + mbase, m_per_w)],
        )

    return fill


def kernel(input):
    s = int(input.shape[0])
    num_experts = _NUM_EXPERTS
    capacity_fp = 2 * s / num_experts
    capacity = int(math.ceil(capacity_fp))
    k = num_experts * capacity // s

    fused = _make_sc_fill(s, num_experts, capacity, 1.0 / k)()
    gates = jax.lax.bitcast_convert_type(fused[: 2 * s], jnp.float32).reshape(2, s)
    dispatch_mask = fused[2 * s :].reshape(num_experts, capacity)
    return (gates, dispatch_mask, jnp.asarray(capacity_fp, dtype=jnp.float32))


# speedup vs baseline: 9.8827x; 9.7685x over previous
import math

import jax
import jax.numpy as jnp
from jax.experimental import pallas as pl

_NUM_EXPERTS = 16


def _fill_kernel(k_inv: float, gates_ref, mask_ref):
    gates_ref[...] = jnp.full(gates_ref.shape, k_inv, dtype=jnp.float32)
    e = jax.lax.broadcasted_iota(jnp.int32, mask_ref.shape, 0)
    c = jax.lax.broadcasted_iota(jnp.int32, mask_ref.shape, 1)
    mask_ref[...] = c * _NUM_EXPERTS + e


def kernel(input):
    s = int(input.shape[0])
    num_experts = _NUM_EXPERTS
    capacity_fp = 2 * s / num_experts
    capacity = int(math.ceil(capacity_fp))
    k = num_experts * capacity // s

    gates, dispatch_mask = pl.pallas_call(
        lambda g, m: _fill_kernel(1.0 / k, g, m),
        out_shape=(
            jax.ShapeDtypeStruct((2, s), jnp.float32),
            jax.ShapeDtypeStruct((num_experts, capacity), jnp.int32),
        ),
    )()
    return (gates, dispatch_mask, jnp.asarray(capacity_fp, dtype=jnp.float32))


# speedup vs baseline: 18.4363x; 1.8655x over previous
import math

import jax
import jax.numpy as jnp
from jax.experimental import pallas as pl

_NUM_EXPERTS = 16


def _fill_kernel(k_inv: float, capacity_fp: float, gates_ref, mask_ref, cap_ref):
    gates_ref[...] = jnp.full(gates_ref.shape, k_inv, dtype=jnp.float32)
    e = jax.lax.broadcasted_iota(jnp.int32, mask_ref.shape, 0)
    c = jax.lax.broadcasted_iota(jnp.int32, mask_ref.shape, 1)
    mask_ref[...] = c * _NUM_EXPERTS + e
    cap_ref[...] = jnp.full(cap_ref.shape, capacity_fp, dtype=jnp.float32)


def kernel(input):
    s = int(input.shape[0])
    num_experts = _NUM_EXPERTS
    capacity_fp = 2 * s / num_experts
    capacity = int(math.ceil(capacity_fp))
    k = num_experts * capacity // s

    gates, dispatch_mask, cap = pl.pallas_call(
        lambda g, m, c: _fill_kernel(1.0 / k, capacity_fp, g, m, c),
        out_shape=(
            jax.ShapeDtypeStruct((2, s), jnp.float32),
            jax.ShapeDtypeStruct((num_experts, capacity), jnp.int32),
            jax.ShapeDtypeStruct((1, 1), jnp.float32),
        ),
    )()
    return (gates, dispatch_mask, cap.reshape(()))
